# Initial kernel scaffold; baseline (speedup 1.0000x reference)
import jax
import jax.numpy as jnp
from jax.experimental import pallas as pl


def kernel(x, edge_index, edge_attr, current_timestep_node_ids, W1, b1, a_src1, a_dst1, We1, a_edge1, W2, b2, a_src2, a_dst2, We2, a_edge2, Wh, bh):
    raise NotImplementedError("write your pallas kernel here")


# speedup vs baseline: 32.4937x; 32.4937x over previous
import jax
import jax.numpy as jnp
from jax import lax
from jax.experimental import pallas as pl
from jax.experimental.pallas import tpu as pltpu
from jax.experimental.pallas import tpu_sc as plsc

N = 100000
E = 1600000
D_T = 16
HEADS = 4

NB = 5
BSZ = 10016
NP = 2 * NB * BSZ
SC_SPAN = NB * BSZ
ROWS_T = BSZ // 16
TW = 136
AW = 136
TRASH = BSZ
CH = 2000
EPT = E // 16
NCH = EPT // CH
GRP = 32

NBLK = 2048
NGRID = (NP + NBLK - 1) // NBLK
EBLK = 4096
EGRID = (E + EBLK - 1) // EBLK


def _node_prep_body(x_ref, w_ref, abd_ref, t_ref, ald_ref):
    h = jnp.dot(x_ref[...], w_ref[...], preferred_element_type=jnp.float32)
    alsd = jnp.dot(h, abd_ref[...], preferred_element_type=jnp.float32)
    z = jnp.zeros((h.shape[0], TW - 132), dtype=jnp.float32)
    t_ref[...] = jnp.concatenate([h, alsd[:, :HEADS], z], axis=1)
    ald_ref[...] = alsd[:, HEADS:]


def _edge_prep_body(ea_ref, ae_ref, o1_ref, o2_ref):
    ale = jnp.dot(ea_ref[...], ae_ref[...], preferred_element_type=jnp.float32)
    o1_ref[...] = ale[:, :HEADS]
    o2_ref[...] = ale[:, HEADS:]


def _mid_body(acc_ref, b1_ref, w2_ref, abd_ref, t_ref, ald_ref):
    acc = acc_ref[...]
    den = acc[:, :HEADS] + 1e-16
    den128 = jnp.broadcast_to(den[:, :, None], (acc.shape[0], HEADS, 32)
                              ).reshape(acc.shape[0], 128)
    agg = acc[:, HEADS:HEADS + 128] / den128 + b1_ref[...]
    h1 = jnp.where(agg > 0, agg, jnp.exp(jnp.minimum(agg, 0.0)) - 1.0)
    h2 = jnp.dot(h1, w2_ref[...], preferred_element_type=jnp.float32)
    alsd = jnp.dot(h2, abd_ref[...], preferred_element_type=jnp.float32)
    z = jnp.zeros((h2.shape[0], TW - 132), dtype=jnp.float32)
    t_ref[...] = jnp.concatenate([h2, alsd[:, :HEADS], z], axis=1)
    ald_ref[...] = alsd[:, HEADS:]


def _head_body(acc_ref, b2_ref, wh_ref, bh_ref, o_ref):
    acc = acc_ref[...]
    g = jnp.zeros((acc.shape[0], 32), dtype=jnp.float32)
    for h in range(HEADS):
        den = acc[:, h:h + 1] + 1e-16
        g = g + acc[:, HEADS + 32 * h:HEADS + 32 * (h + 1)] / den
    g = g * (1.0 / HEADS) + b2_ref[...]
    o_ref[...] = jnp.dot(g, wh_ref[...], preferred_element_type=jnp.float32) \
        + bh_ref[...]


def _edge_pass_body(dst_h, src_h, ale_h, t_h, ald_h, zeros_h, acc_h,
                    acc_sh, dstb, srcb, aleb, eoffL, srcL, dstL,
                    ale0L, ale1L, ale2L, ale3L,
                    idxg, idxs, idxd, rowsb, outb, aldr):
    aleLs = (ale0L, ale1L, ale2L, ale3L)
    c = lax.axis_index("c")
    s = lax.axis_index("s")
    sc_base = c * SC_SPAN
    my_rows = s * ROWS_T
    lanes = lax.iota(jnp.int32, 16)

    def bucket_body(b, carry):
        lo = sc_base + b * BSZ
        pltpu.sync_copy(zeros_h, acc_sh.at[pl.ds(my_rows, ROWS_T)])
        plsc.subcore_barrier()

        def chunk_body(ci, carry2):
            e0 = s * EPT + ci * CH

            pltpu.sync_copy(dst_h.at[pl.ds(e0, CH)], dstb)
            pltpu.sync_copy(src_h.at[pl.ds(e0, CH)], srcb)
            pltpu.sync_copy(ale_h.at[pl.ds(e0 * 4, CH * 4)], aleb)

            def filt(i, cnt):
                d = dstb[pl.ds(i * 16, 16)]
                eo = d - lo
                m = (eo >= 0) & (eo < BSZ)
                plsc.store_compressed(eoffL.at[pl.ds(cnt, 16)],
                                      plsc.bitcast(eo, jnp.float32), mask=m)
                plsc.store_compressed(dstL.at[pl.ds(cnt, 16)],
                                      plsc.bitcast(d, jnp.float32), mask=m)
                plsc.store_compressed(
                    srcL.at[pl.ds(cnt, 16)],
                    plsc.bitcast(srcb[pl.ds(i * 16, 16)], jnp.float32), mask=m)
                for h in range(HEADS):
                    av = plsc.load_gather(
                        aleb, [(lanes + i * 16) * 4 + h])
                    plsc.store_compressed(aleLs[h].at[pl.ds(cnt, 16)], av, mask=m)
                return cnt + jnp.sum(m.astype(jnp.int32))
            cnt = lax.fori_loop(0, CH // 16, filt, jnp.int32(0))

            tr = plsc.bitcast(jnp.full((16,), TRASH, jnp.int32), jnp.float32)
            zi = plsc.bitcast(jnp.zeros((16,), jnp.int32), jnp.float32)
            eoffL[pl.ds(cnt, 16)] = tr
            eoffL[pl.ds(cnt + 16, 16)] = tr
            srcL[pl.ds(cnt, 16)] = zi
            srcL[pl.ds(cnt + 16, 16)] = zi
            dstL[pl.ds(cnt, 16)] = zi
            dstL[pl.ds(cnt + 16, 16)] = zi
            ngroups = lax.shift_right_logical(cnt + (GRP - 1), 5)

            def grp(g, carry3):
                gb = g * GRP
                for q in range(GRP // 16):
                    idxg[pl.ds(q * 16, 16)] = plsc.bitcast(
                        srcL[pl.ds(gb + q * 16, 16)], jnp.int32)
                    idxs[pl.ds(q * 16, 16)] = plsc.bitcast(
                        eoffL[pl.ds(gb + q * 16, 16)], jnp.int32)
                    idxd[pl.ds(q * 16, 16)] = plsc.bitcast(
                        dstL[pl.ds(gb + q * 16, 16)], jnp.int32)
                pltpu.sync_copy(t_h.at[idxg], rowsb)
                pltpu.sync_copy(ald_h.at[idxd], aldr)

                for half in range(GRP // 16):
                    hb = gb + half * 16
                    eoffv = plsc.bitcast(eoffL[pl.ds(hb, 16)], jnp.int32)
                    lane_h = lanes + half * 16
                    exs = []
                    for h in range(HEADS):
                        alev = aleLs[h][pl.ds(hb, 16)]
                        alsv = plsc.load_gather(
                            rowsb,
                            [lane_h, jnp.full((16,), 128 + h, jnp.int32)])
                        aldvv = plsc.load_gather(
                            aldr, [lane_h, jnp.full((16,), h, jnp.int32)])
                        lg = alsv + aldvv + alev
                        lg = jnp.maximum(lg, 0.2 * lg)
                        ex = jnp.exp(lg)
                        plsc.store_scatter(
                            outb, [lane_h, jnp.full((16,), h, jnp.int32)], ex)
                        exs.append(ex)
                    for e in range(16):
                        ei = half * 16 + e
                        for h in range(HEADS):
                            w = lax.broadcast(exs[h][e], (16,))
                            for r in range(2):
                                col = h * 32 + r * 16
                                hv = rowsb[ei, pl.ds(col, 16)]
                                outb[ei, pl.ds(HEADS + col, 16)] = hv * w
                pltpu.sync_copy(outb, acc_sh.at[idxs], add=True)
                return carry3
            lax.fori_loop(0, ngroups, grp, 0)
            return carry2
        lax.fori_loop(0, NCH, chunk_body, 0)

        plsc.subcore_barrier()
        pltpu.sync_copy(acc_sh.at[pl.ds(my_rows, ROWS_T)],
                        acc_h.at[pl.ds(lo + my_rows, ROWS_T)])
        plsc.subcore_barrier()
        return carry
    lax.fori_loop(0, NB, bucket_body, 0)


def _pad8(a):
    return jnp.concatenate([a, jnp.zeros_like(a)], axis=1)


def _edge_pass(dst, src, ale, t, ald, zeros):
    mesh = plsc.VectorSubcoreMesh(core_axis_name="c", subcore_axis_name="s")
    return pl.kernel(
        _edge_pass_body,
        out_type=jax.ShapeDtypeStruct((NP, AW), jnp.float32),
        mesh=mesh,
        compiler_params=pltpu.CompilerParams(needs_layout_passes=False, use_tc_tiling_on_sc=False),
        scratch_types=[
            pltpu.VMEM_SHARED((BSZ + 8, AW), jnp.float32),
            pltpu.VMEM((CH,), jnp.int32),
            pltpu.VMEM((CH,), jnp.int32),
            pltpu.VMEM((CH * 4,), jnp.float32),
            pltpu.VMEM((CH + 48,), jnp.float32),
            pltpu.VMEM((CH + 48,), jnp.float32),
            pltpu.VMEM((CH + 48,), jnp.float32),
            pltpu.VMEM((CH + 48,), jnp.float32),
            pltpu.VMEM((CH + 48,), jnp.float32),
            pltpu.VMEM((CH + 48,), jnp.float32),
            pltpu.VMEM((CH + 48,), jnp.float32),
            pltpu.VMEM((GRP,), jnp.int32),
            pltpu.VMEM((GRP,), jnp.int32),
            pltpu.VMEM((GRP,), jnp.int32),
            pltpu.VMEM((GRP, TW), jnp.float32),
            pltpu.VMEM((GRP, AW), jnp.float32),
            pltpu.VMEM((GRP, 8), jnp.float32),
        ],
    )(dst, src, ale, t, ald, zeros)


def kernel(x, edge_index, edge_attr, current_timestep_node_ids,
           W1, b1, a_src1, a_dst1, We1, a_edge1,
           W2, b2, a_src2, a_dst2, We2, a_edge2,
           Wh, bh):
    f32 = jnp.float32
    src = edge_index[0].astype(jnp.int32)
    dst = edge_index[1].astype(jnp.int32)

    W1p = jnp.concatenate([W1[32:48], W1[0:32]], axis=0)

    def blockdiag(a):
        hd = a.shape[0] * a.shape[1]
        m = jnp.zeros((hd, a.shape[0]), f32)
        for h in range(a.shape[0]):
            m = m.at[h * a.shape[1]:(h + 1) * a.shape[1], h].set(a[h])
        return m
    Abd1 = jnp.concatenate([blockdiag(a_src1), blockdiag(a_dst1)], axis=1)
    Abd2 = jnp.concatenate([blockdiag(a_src2), blockdiag(a_dst2)], axis=1)
    Ae1 = (We1.reshape(16, HEADS, 32) * a_edge1[None]).sum(-1)
    Ae2 = (We2.reshape(16, HEADS, 32) * a_edge2[None]).sum(-1)
    AeB = jnp.concatenate([Ae1, Ae2], axis=1)

    full = lambda shp: pl.BlockSpec(shp, lambda i: (0, 0))
    rows = lambda w: pl.BlockSpec((NBLK, w), lambda i: (i, 0))
    erows = lambda w: pl.BlockSpec((EBLK, w), lambda i: (i, 0))

    t1, ald1 = pl.pallas_call(
        _node_prep_body,
        grid=(NGRID,),
        in_specs=[rows(48), full((48, 128)), full((128, 2 * HEADS))],
        out_specs=(rows(TW), rows(HEADS)),
        out_shape=(jax.ShapeDtypeStruct((NP, TW), f32),
                   jax.ShapeDtypeStruct((NP, HEADS), f32)),
    )(x, W1p, Abd1)

    ale1, ale2 = pl.pallas_call(
        _edge_prep_body,
        grid=(EGRID,),
        in_specs=[erows(16), full((16, 2 * HEADS))],
        out_specs=(erows(HEADS), erows(HEADS)),
        out_shape=(jax.ShapeDtypeStruct((E, HEADS), f32),
                   jax.ShapeDtypeStruct((E, HEADS), f32)),
    )(edge_attr, AeB)

    zeros = jnp.zeros((ROWS_T, AW), f32)
    acc1 = _edge_pass(dst, src, ale1.reshape(-1), t1, _pad8(ald1), zeros)

    t2, ald2 = pl.pallas_call(
        _mid_body,
        grid=(NGRID,),
        in_specs=[rows(AW), pl.BlockSpec((1, 128), lambda i: (0, 0)),
                  full((128, 128)), full((128, 2 * HEADS))],
        out_specs=(rows(TW), rows(HEADS)),
        out_shape=(jax.ShapeDtypeStruct((NP, TW), f32),
                   jax.ShapeDtypeStruct((NP, HEADS), f32)),
    )(acc1, b1.reshape(1, 128), W2, Abd2)

    acc2 = _edge_pass(dst, src, ale2.reshape(-1), t2, _pad8(ald2), zeros)

    pred = pl.pallas_call(
        _head_body,
        grid=(NGRID,),
        in_specs=[rows(AW), pl.BlockSpec((1, 32), lambda i: (0, 0)),
                  full((32, 1)), pl.BlockSpec((1, 1), lambda i: (0, 0))],
        out_specs=rows(1),
        out_shape=jax.ShapeDtypeStruct((NP, 1), f32),
    )(acc2, b2.reshape(1, 32), Wh, bh.reshape(1, 1))

    return pred[:N]


# speedup vs baseline: 35.1329x; 1.0812x over previous
import jax
import jax.numpy as jnp
from jax import lax
from jax.experimental import pallas as pl
from jax.experimental.pallas import tpu as pltpu
from jax.experimental.pallas import tpu_sc as plsc

N = 100000
E = 1600000
D_T = 16
HEADS = 4

NB = 5
BSZ = 10016
NP = 2 * NB * BSZ
SC_SPAN = NB * BSZ
ROWS_T = BSZ // 16
TW = 136
AW = 136
TRASH = BSZ
CH = 2000
EPT = E // 16
NCH = EPT // CH
GRP = 32

NBLK = 2048
NGRID = (NP + NBLK - 1) // NBLK
EBLK = 4096
EGRID = (E + EBLK - 1) // EBLK


def _node_prep_body(x_ref, w_ref, abd_ref, t_ref, ald_ref):
    h = jnp.dot(x_ref[...], w_ref[...], preferred_element_type=jnp.float32)
    alsd = jnp.dot(h, abd_ref[...], preferred_element_type=jnp.float32)
    z = jnp.zeros((h.shape[0], TW - 132), dtype=jnp.float32)
    t_ref[...] = jnp.concatenate([h, alsd[:, :HEADS], z], axis=1)
    ald_ref[...] = alsd[:, HEADS:]


def _edge_prep_body(ea_ref, ae_ref, o1_ref, o2_ref):
    ale = jnp.dot(ea_ref[...], ae_ref[...], preferred_element_type=jnp.float32)
    o1_ref[...] = ale[:, :HEADS]
    o2_ref[...] = ale[:, HEADS:]


def _mid_body(acc_ref, b1_ref, w2_ref, abd_ref, t_ref, ald_ref):
    acc = acc_ref[...]
    den = acc[:, :HEADS] + 1e-16
    den128 = jnp.broadcast_to(den[:, :, None], (acc.shape[0], HEADS, 32)
                              ).reshape(acc.shape[0], 128)
    agg = acc[:, HEADS:HEADS + 128] / den128 + b1_ref[...]
    h1 = jnp.where(agg > 0, agg, jnp.exp(jnp.minimum(agg, 0.0)) - 1.0)
    h2 = jnp.dot(h1, w2_ref[...], preferred_element_type=jnp.float32)
    alsd = jnp.dot(h2, abd_ref[...], preferred_element_type=jnp.float32)
    z = jnp.zeros((h2.shape[0], TW - 132), dtype=jnp.float32)
    t_ref[...] = jnp.concatenate([h2, alsd[:, :HEADS], z], axis=1)
    ald_ref[...] = alsd[:, HEADS:]


def _head_body(acc_ref, b2_ref, wh_ref, bh_ref, o_ref):
    acc = acc_ref[...]
    g = jnp.zeros((acc.shape[0], 32), dtype=jnp.float32)
    for h in range(HEADS):
        den = acc[:, h:h + 1] + 1e-16
        g = g + acc[:, HEADS + 32 * h:HEADS + 32 * (h + 1)] / den
    g = g * (1.0 / HEADS) + b2_ref[...]
    o_ref[...] = jnp.dot(g, wh_ref[...], preferred_element_type=jnp.float32) \
        + bh_ref[...]


def _edge_pass_body(dst_h, src_h, ale_h, t_h, ald_h, zeros_h, acc_h,
                    acc_sh, dstb, srcb, aleb, eoffL, srcL, dstL,
                    ale0L, ale1L, ale2L, ale3L,
                    idxgA, idxs, idxdA, rowsbA, outb, aldrA,
                    idxgB, idxdB, rowsbB, aldrB, semA, semB):
    aleLs = (ale0L, ale1L, ale2L, ale3L)
    c = lax.axis_index("c")
    s = lax.axis_index("s")
    sc_base = c * SC_SPAN
    my_rows = s * ROWS_T
    lanes = lax.iota(jnp.int32, 16)

    def bucket_body(b, carry):
        lo = sc_base + b * BSZ
        pltpu.sync_copy(zeros_h, acc_sh.at[pl.ds(my_rows, ROWS_T)])
        plsc.subcore_barrier()

        def chunk_body(ci, carry2):
            e0 = s * EPT + ci * CH

            pltpu.sync_copy(dst_h.at[pl.ds(e0, CH)], dstb)
            pltpu.sync_copy(src_h.at[pl.ds(e0, CH)], srcb)
            pltpu.sync_copy(ale_h.at[pl.ds(e0 * 4, CH * 4)], aleb)

            def filt(i, cnt):
                d = dstb[pl.ds(i * 16, 16)]
                eo = d - lo
                m = (eo >= 0) & (eo < BSZ)
                plsc.store_compressed(eoffL.at[pl.ds(cnt, 16)],
                                      plsc.bitcast(eo, jnp.float32), mask=m)
                plsc.store_compressed(dstL.at[pl.ds(cnt, 16)],
                                      plsc.bitcast(d, jnp.float32), mask=m)
                plsc.store_compressed(
                    srcL.at[pl.ds(cnt, 16)],
                    plsc.bitcast(srcb[pl.ds(i * 16, 16)], jnp.float32), mask=m)
                for h in range(HEADS):
                    av = plsc.load_gather(
                        aleb, [(lanes + i * 16) * 4 + h])
                    plsc.store_compressed(aleLs[h].at[pl.ds(cnt, 16)], av, mask=m)
                return cnt + jnp.sum(m.astype(jnp.int32))
            cnt = lax.fori_loop(0, CH // 16, filt, jnp.int32(0))

            tr = plsc.bitcast(jnp.full((16,), TRASH, jnp.int32), jnp.float32)
            zi = plsc.bitcast(jnp.zeros((16,), jnp.int32), jnp.float32)
            eoffL[pl.ds(cnt, 16)] = tr
            eoffL[pl.ds(cnt + 16, 16)] = tr
            srcL[pl.ds(cnt, 16)] = zi
            srcL[pl.ds(cnt + 16, 16)] = zi
            dstL[pl.ds(cnt, 16)] = zi
            dstL[pl.ds(cnt + 16, 16)] = zi
            ngroups = lax.shift_right_logical(cnt + (GRP - 1), 5)

            bufs = ((idxgA, idxdA, rowsbA, aldrA, semA),
                    (idxgB, idxdB, rowsbB, aldrB, semB))

            def copy_idx(g, ig, idv):
                gb = g * GRP
                for q in range(GRP // 16):
                    ig[pl.ds(q * 16, 16)] = plsc.bitcast(
                        srcL[pl.ds(gb + q * 16, 16)], jnp.int32)
                    idv[pl.ds(q * 16, 16)] = plsc.bitcast(
                        dstL[pl.ds(gb + q * 16, 16)], jnp.int32)

            def fire(buf):
                ig, idv, rb, ar, sem = buf
                pltpu.async_copy(t_h.at[ig], rb, sem)
                pltpu.async_copy(ald_h.at[idv], ar, sem)

            def waitg(buf):
                ig, idv, rb, ar, sem = buf
                pltpu.make_async_copy(t_h.at[ig], rb, sem).wait()
                pltpu.make_async_copy(ald_h.at[idv], ar, sem).wait()

            def compute(g, buf):
                ig, idv, rb, ar, sem = buf
                gb = g * GRP
                for q in range(GRP // 16):
                    idxs[pl.ds(q * 16, 16)] = plsc.bitcast(
                        eoffL[pl.ds(gb + q * 16, 16)], jnp.int32)
                for half in range(GRP // 16):
                    hb = gb + half * 16
                    lane_h = lanes + half * 16
                    exs = []
                    for h in range(HEADS):
                        alev = aleLs[h][pl.ds(hb, 16)]
                        alsv = plsc.load_gather(
                            rb, [lane_h, jnp.full((16,), 128 + h, jnp.int32)])
                        aldvv = plsc.load_gather(
                            ar, [lane_h, jnp.full((16,), h, jnp.int32)])
                        lg = alsv + aldvv + alev
                        lg = jnp.maximum(lg, 0.2 * lg)
                        ex = jnp.exp(lg)
                        plsc.store_scatter(
                            outb, [lane_h, jnp.full((16,), h, jnp.int32)], ex)
                        exs.append(ex)
                    for e in range(16):
                        ei = half * 16 + e
                        for h in range(HEADS):
                            w = lax.broadcast(exs[h][e], (16,))
                            for r in range(2):
                                col = h * 32 + r * 16
                                hv = rb[ei, pl.ds(col, 16)]
                                outb[ei, pl.ds(HEADS + col, 16)] = hv * w
                pltpu.sync_copy(outb, acc_sh.at[idxs], add=True)

            @pl.when(ngroups > 0)
            def _():
                copy_idx(0, idxgA, idxdA)
                fire(bufs[0])

            def gp_body(gp, carry3):
                g0 = gp * 2
                g1 = g0 + 1

                @pl.when(g1 < ngroups)
                def _():
                    copy_idx(g1, idxgB, idxdB)
                    fire(bufs[1])
                waitg(bufs[0])
                compute(g0, bufs[0])

                @pl.when(g1 < ngroups)
                def _():
                    @pl.when(g1 + 1 < ngroups)
                    def _():
                        copy_idx(g1 + 1, idxgA, idxdA)
                        fire(bufs[0])
                    waitg(bufs[1])
                    compute(g1, bufs[1])
                return carry3
            lax.fori_loop(0, lax.shift_right_logical(ngroups + 1, 1),
                          gp_body, 0)
            return carry2
        lax.fori_loop(0, NCH, chunk_body, 0)

        plsc.subcore_barrier()
        pltpu.sync_copy(acc_sh.at[pl.ds(my_rows, ROWS_T)],
                        acc_h.at[pl.ds(lo + my_rows, ROWS_T)])
        plsc.subcore_barrier()
        return carry
    lax.fori_loop(0, NB, bucket_body, 0)


def _pad8(a):
    return jnp.concatenate([a, jnp.zeros_like(a)], axis=1)


def _edge_pass(dst, src, ale, t, ald, zeros):
    mesh = plsc.VectorSubcoreMesh(core_axis_name="c", subcore_axis_name="s")
    return pl.kernel(
        _edge_pass_body,
        out_type=jax.ShapeDtypeStruct((NP, AW), jnp.float32),
        mesh=mesh,
        compiler_params=pltpu.CompilerParams(needs_layout_passes=False, use_tc_tiling_on_sc=False),
        scratch_types=[
            pltpu.VMEM_SHARED((BSZ + 8, AW), jnp.float32),
            pltpu.VMEM((CH,), jnp.int32),
            pltpu.VMEM((CH,), jnp.int32),
            pltpu.VMEM((CH * 4,), jnp.float32),
            pltpu.VMEM((CH + 48,), jnp.float32),
            pltpu.VMEM((CH + 48,), jnp.float32),
            pltpu.VMEM((CH + 48,), jnp.float32),
            pltpu.VMEM((CH + 48,), jnp.float32),
            pltpu.VMEM((CH + 48,), jnp.float32),
            pltpu.VMEM((CH + 48,), jnp.float32),
            pltpu.VMEM((CH + 48,), jnp.float32),
            pltpu.VMEM((GRP,), jnp.int32),
            pltpu.VMEM((GRP,), jnp.int32),
            pltpu.VMEM((GRP,), jnp.int32),
            pltpu.VMEM((GRP, TW), jnp.float32),
            pltpu.VMEM((GRP, AW), jnp.float32),
            pltpu.VMEM((GRP, 8), jnp.float32),
            pltpu.VMEM((GRP,), jnp.int32),
            pltpu.VMEM((GRP,), jnp.int32),
            pltpu.VMEM((GRP, TW), jnp.float32),
            pltpu.VMEM((GRP, 8), jnp.float32),
            pltpu.SemaphoreType.DMA,
            pltpu.SemaphoreType.DMA,
        ],
    )(dst, src, ale, t, ald, zeros)


def kernel(x, edge_index, edge_attr, current_timestep_node_ids,
           W1, b1, a_src1, a_dst1, We1, a_edge1,
           W2, b2, a_src2, a_dst2, We2, a_edge2,
           Wh, bh):
    f32 = jnp.float32
    src = edge_index[0].astype(jnp.int32)
    dst = edge_index[1].astype(jnp.int32)

    W1p = jnp.concatenate([W1[32:48], W1[0:32]], axis=0)

    def blockdiag(a):
        hd = a.shape[0] * a.shape[1]
        m = jnp.zeros((hd, a.shape[0]), f32)
        for h in range(a.shape[0]):
            m = m.at[h * a.shape[1]:(h + 1) * a.shape[1], h].set(a[h])
        return m
    Abd1 = jnp.concatenate([blockdiag(a_src1), blockdiag(a_dst1)], axis=1)
    Abd2 = jnp.concatenate([blockdiag(a_src2), blockdiag(a_dst2)], axis=1)
    Ae1 = (We1.reshape(16, HEADS, 32) * a_edge1[None]).sum(-1)
    Ae2 = (We2.reshape(16, HEADS, 32) * a_edge2[None]).sum(-1)
    AeB = jnp.concatenate([Ae1, Ae2], axis=1)

    full = lambda shp: pl.BlockSpec(shp, lambda i: (0, 0))
    rows = lambda w: pl.BlockSpec((NBLK, w), lambda i: (i, 0))
    erows = lambda w: pl.BlockSpec((EBLK, w), lambda i: (i, 0))

    t1, ald1 = pl.pallas_call(
        _node_prep_body,
        grid=(NGRID,),
        in_specs=[rows(48), full((48, 128)), full((128, 2 * HEADS))],
        out_specs=(rows(TW), rows(HEADS)),
        out_shape=(jax.ShapeDtypeStruct((NP, TW), f32),
                   jax.ShapeDtypeStruct((NP, HEADS), f32)),
    )(x, W1p, Abd1)

    ale1, ale2 = pl.pallas_call(
        _edge_prep_body,
        grid=(EGRID,),
        in_specs=[erows(16), full((16, 2 * HEADS))],
        out_specs=(erows(HEADS), erows(HEADS)),
        out_shape=(jax.ShapeDtypeStruct((E, HEADS), f32),
                   jax.ShapeDtypeStruct((E, HEADS), f32)),
    )(edge_attr, AeB)

    zeros = jnp.zeros((ROWS_T, AW), f32)
    acc1 = _edge_pass(dst, src, ale1.reshape(-1), t1, _pad8(ald1), zeros)

    t2, ald2 = pl.pallas_call(
        _mid_body,
        grid=(NGRID,),
        in_specs=[rows(AW), pl.BlockSpec((1, 128), lambda i: (0, 0)),
                  full((128, 128)), full((128, 2 * HEADS))],
        out_specs=(rows(TW), rows(HEADS)),
        out_shape=(jax.ShapeDtypeStruct((NP, TW), f32),
                   jax.ShapeDtypeStruct((NP, HEADS), f32)),
    )(acc1, b1.reshape(1, 128), W2, Abd2)

    acc2 = _edge_pass(dst, src, ale2.reshape(-1), t2, _pad8(ald2), zeros)

    pred = pl.pallas_call(
        _head_body,
        grid=(NGRID,),
        in_specs=[rows(AW), pl.BlockSpec((1, 32), lambda i: (0, 0)),
                  full((32, 1)), pl.BlockSpec((1, 1), lambda i: (0, 0))],
        out_specs=rows(1),
        out_shape=jax.ShapeDtypeStruct((NP, 1), f32),
    )(acc2, b2.reshape(1, 32), Wh, bh.reshape(1, 1))

    return pred[:N]


# speedup vs baseline: 35.2087x; 1.0022x over previous
import jax
import jax.numpy as jnp
from jax import lax
from jax.experimental import pallas as pl
from jax.experimental.pallas import tpu as pltpu
from jax.experimental.pallas import tpu_sc as plsc

N = 100000
E = 1600000
D_T = 16
HEADS = 4

NB = 5
BSZ = 10016
NP = 2 * NB * BSZ
SC_SPAN = NB * BSZ
ROWS_T = BSZ // 16
TW = 136
AW = 136
TRASH = BSZ
CH = 2000
EPT = E // 16
NCH = EPT // CH
GRP = 32

NBLK = 2048
NGRID = (NP + NBLK - 1) // NBLK
EBLK = 4096
EGRID = (E + EBLK - 1) // EBLK


def _node_prep_body(x_ref, w_ref, abd_ref, t_ref, ald_ref):
    h = jnp.dot(x_ref[...], w_ref[...], preferred_element_type=jnp.float32)
    alsd = jnp.dot(h, abd_ref[...], preferred_element_type=jnp.float32)
    z = jnp.zeros((h.shape[0], TW - 132), dtype=jnp.float32)
    t_ref[...] = jnp.concatenate([h, alsd[:, :HEADS], z], axis=1)
    ald_ref[...] = alsd[:, HEADS:]


def _edge_prep_body(ea_ref, ae_ref, o1_ref, o2_ref):
    ale = jnp.dot(ea_ref[...], ae_ref[...], preferred_element_type=jnp.float32)
    o1_ref[...] = ale[:, :HEADS]
    o2_ref[...] = ale[:, HEADS:]


def _mid_body(acc_ref, b1_ref, w2_ref, abd_ref, t_ref, ald_ref):
    acc = acc_ref[...]
    den = acc[:, :HEADS] + 1e-16
    den128 = jnp.broadcast_to(den[:, :, None], (acc.shape[0], HEADS, 32)
                              ).reshape(acc.shape[0], 128)
    agg = acc[:, HEADS:HEADS + 128] / den128 + b1_ref[...]
    h1 = jnp.where(agg > 0, agg, jnp.exp(jnp.minimum(agg, 0.0)) - 1.0)
    h2 = jnp.dot(h1, w2_ref[...], preferred_element_type=jnp.float32)
    alsd = jnp.dot(h2, abd_ref[...], preferred_element_type=jnp.float32)
    z = jnp.zeros((h2.shape[0], TW - 132), dtype=jnp.float32)
    t_ref[...] = jnp.concatenate([h2, alsd[:, :HEADS], z], axis=1)
    ald_ref[...] = alsd[:, HEADS:]


def _head_body(acc_ref, b2_ref, wh_ref, bh_ref, o_ref):
    acc = acc_ref[...]
    g = jnp.zeros((acc.shape[0], 32), dtype=jnp.float32)
    for h in range(HEADS):
        den = acc[:, h:h + 1] + 1e-16
        g = g + acc[:, HEADS + 32 * h:HEADS + 32 * (h + 1)] / den
    g = g * (1.0 / HEADS) + b2_ref[...]
    o_ref[...] = jnp.dot(g, wh_ref[...], preferred_element_type=jnp.float32) \
        + bh_ref[...]


def _edge_pass_body(dst_h, src_h, ale_h, t_h, ald_h, zeros_h, acc_h,
                    acc_sh, dstb, srcb, aleb, eoffL, srcL, dstL,
                    ale0L, ale1L, ale2L, ale3L,
                    idxgA, idxs, idxdA, rowsbA, outb, aldrA,
                    idxgB, idxdB, rowsbB, aldrB, semA, semB, semC):
    aleLs = (ale0L, ale1L, ale2L, ale3L)
    c = lax.axis_index("c")
    s = lax.axis_index("s")
    sc_base = c * SC_SPAN
    my_rows = s * ROWS_T
    lanes = lax.iota(jnp.int32, 16)

    def bucket_body(b, carry):
        lo = sc_base + b * BSZ
        pltpu.sync_copy(zeros_h, acc_sh.at[pl.ds(my_rows, ROWS_T)])
        plsc.subcore_barrier()

        def chunk_body(ci, carry2):
            e0 = s * EPT + ci * CH

            d1 = pltpu.async_copy(dst_h.at[pl.ds(e0, CH)], dstb, semC)
            d2 = pltpu.async_copy(src_h.at[pl.ds(e0, CH)], srcb, semC)
            d3 = pltpu.async_copy(ale_h.at[pl.ds(e0 * 4, CH * 4)], aleb, semC)
            d1.wait()
            d2.wait()
            d3.wait()

            def filt(i, cnt):
                d = dstb[pl.ds(i * 16, 16)]
                eo = d - lo
                m = (eo >= 0) & (eo < BSZ)
                plsc.store_compressed(eoffL.at[pl.ds(cnt, 16)],
                                      plsc.bitcast(eo, jnp.float32), mask=m)
                plsc.store_compressed(dstL.at[pl.ds(cnt, 16)],
                                      plsc.bitcast(d, jnp.float32), mask=m)
                plsc.store_compressed(
                    srcL.at[pl.ds(cnt, 16)],
                    plsc.bitcast(srcb[pl.ds(i * 16, 16)], jnp.float32), mask=m)
                for h in range(HEADS):
                    av = plsc.load_gather(
                        aleb, [(lanes + i * 16) * 4 + h])
                    plsc.store_compressed(aleLs[h].at[pl.ds(cnt, 16)], av, mask=m)
                return cnt + jnp.sum(m.astype(jnp.int32))
            cnt = lax.fori_loop(0, CH // 16, filt, jnp.int32(0))

            tr = plsc.bitcast(jnp.full((16,), TRASH, jnp.int32), jnp.float32)
            zi = plsc.bitcast(jnp.zeros((16,), jnp.int32), jnp.float32)
            eoffL[pl.ds(cnt, 16)] = tr
            eoffL[pl.ds(cnt + 16, 16)] = tr
            srcL[pl.ds(cnt, 16)] = zi
            srcL[pl.ds(cnt + 16, 16)] = zi
            dstL[pl.ds(cnt, 16)] = zi
            dstL[pl.ds(cnt + 16, 16)] = zi
            ngroups = lax.shift_right_logical(cnt + (GRP - 1), 5)

            bufs = ((idxgA, idxdA, rowsbA, aldrA, semA),
                    (idxgB, idxdB, rowsbB, aldrB, semB))

            def copy_idx(g, ig, idv):
                gb = g * GRP
                for q in range(GRP // 16):
                    ig[pl.ds(q * 16, 16)] = plsc.bitcast(
                        srcL[pl.ds(gb + q * 16, 16)], jnp.int32)
                    idv[pl.ds(q * 16, 16)] = plsc.bitcast(
                        dstL[pl.ds(gb + q * 16, 16)], jnp.int32)

            def fire(buf):
                ig, idv, rb, ar, sem = buf
                pltpu.async_copy(t_h.at[ig], rb, sem)
                pltpu.async_copy(ald_h.at[idv], ar, sem)

            def waitg(buf):
                ig, idv, rb, ar, sem = buf
                pltpu.make_async_copy(t_h.at[ig], rb, sem).wait()
                pltpu.make_async_copy(ald_h.at[idv], ar, sem).wait()

            def compute(g, buf):
                ig, idv, rb, ar, sem = buf
                gb = g * GRP
                for q in range(GRP // 16):
                    idxs[pl.ds(q * 16, 16)] = plsc.bitcast(
                        eoffL[pl.ds(gb + q * 16, 16)], jnp.int32)
                for half in range(GRP // 16):
                    hb = gb + half * 16
                    lane_h = lanes + half * 16
                    exs = []
                    for h in range(HEADS):
                        alev = aleLs[h][pl.ds(hb, 16)]
                        alsv = plsc.load_gather(
                            rb, [lane_h, jnp.full((16,), 128 + h, jnp.int32)])
                        aldvv = plsc.load_gather(
                            ar, [lane_h, jnp.full((16,), h, jnp.int32)])
                        lg = alsv + aldvv + alev
                        lg = jnp.maximum(lg, 0.2 * lg)
                        ex = jnp.exp(lg)
                        plsc.store_scatter(
                            outb, [lane_h, jnp.full((16,), h, jnp.int32)], ex)
                        exs.append(ex)
                    for e in range(16):
                        ei = half * 16 + e
                        for h in range(HEADS):
                            w = lax.broadcast(exs[h][e], (16,))
                            for r in range(2):
                                col = h * 32 + r * 16
                                hv = rb[ei, pl.ds(col, 16)]
                                outb[ei, pl.ds(HEADS + col, 16)] = hv * w
                pltpu.sync_copy(outb, acc_sh.at[idxs], add=True)

            @pl.when(ngroups > 0)
            def _():
                copy_idx(0, idxgA, idxdA)
                fire(bufs[0])

            def gp_body(gp, carry3):
                g0 = gp * 2
                g1 = g0 + 1

                @pl.when(g1 < ngroups)
                def _():
                    copy_idx(g1, idxgB, idxdB)
                    fire(bufs[1])
                waitg(bufs[0])
                compute(g0, bufs[0])

                @pl.when(g1 < ngroups)
                def _():
                    @pl.when(g1 + 1 < ngroups)
                    def _():
                        copy_idx(g1 + 1, idxgA, idxdA)
                        fire(bufs[0])
                    waitg(bufs[1])
                    compute(g1, bufs[1])
                return carry3
            lax.fori_loop(0, lax.shift_right_logical(ngroups + 1, 1),
                          gp_body, 0)
            return carry2
        lax.fori_loop(0, NCH, chunk_body, 0)

        plsc.subcore_barrier()
        pltpu.sync_copy(acc_sh.at[pl.ds(my_rows, ROWS_T)],
                        acc_h.at[pl.ds(lo + my_rows, ROWS_T)])
        plsc.subcore_barrier()
        return carry
    lax.fori_loop(0, NB, bucket_body, 0)


def _pad8(a):
    return jnp.concatenate([a, jnp.zeros_like(a)], axis=1)


def _edge_pass(dst, src, ale, t, ald, zeros):
    mesh = plsc.VectorSubcoreMesh(core_axis_name="c", subcore_axis_name="s")
    return pl.kernel(
        _edge_pass_body,
        out_type=jax.ShapeDtypeStruct((NP, AW), jnp.float32),
        mesh=mesh,
        compiler_params=pltpu.CompilerParams(needs_layout_passes=False, use_tc_tiling_on_sc=False),
        scratch_types=[
            pltpu.VMEM_SHARED((BSZ + 8, AW), jnp.float32),
            pltpu.VMEM((CH,), jnp.int32),
            pltpu.VMEM((CH,), jnp.int32),
            pltpu.VMEM((CH * 4,), jnp.float32),
            pltpu.VMEM((CH + 48,), jnp.float32),
            pltpu.VMEM((CH + 48,), jnp.float32),
            pltpu.VMEM((CH + 48,), jnp.float32),
            pltpu.VMEM((CH + 48,), jnp.float32),
            pltpu.VMEM((CH + 48,), jnp.float32),
            pltpu.VMEM((CH + 48,), jnp.float32),
            pltpu.VMEM((CH + 48,), jnp.float32),
            pltpu.VMEM((GRP,), jnp.int32),
            pltpu.VMEM((GRP,), jnp.int32),
            pltpu.VMEM((GRP,), jnp.int32),
            pltpu.VMEM((GRP, TW), jnp.float32),
            pltpu.VMEM((GRP, AW), jnp.float32),
            pltpu.VMEM((GRP, 8), jnp.float32),
            pltpu.VMEM((GRP,), jnp.int32),
            pltpu.VMEM((GRP,), jnp.int32),
            pltpu.VMEM((GRP, TW), jnp.float32),
            pltpu.VMEM((GRP, 8), jnp.float32),
            pltpu.SemaphoreType.DMA,
            pltpu.SemaphoreType.DMA,
            pltpu.SemaphoreType.DMA,
        ],
    )(dst, src, ale, t, ald, zeros)


def kernel(x, edge_index, edge_attr, current_timestep_node_ids,
           W1, b1, a_src1, a_dst1, We1, a_edge1,
           W2, b2, a_src2, a_dst2, We2, a_edge2,
           Wh, bh):
    f32 = jnp.float32
    src = edge_index[0].astype(jnp.int32)
    dst = edge_index[1].astype(jnp.int32)

    W1p = jnp.concatenate([W1[32:48], W1[0:32]], axis=0)

    def blockdiag(a):
        hd = a.shape[0] * a.shape[1]
        m = jnp.zeros((hd, a.shape[0]), f32)
        for h in range(a.shape[0]):
            m = m.at[h * a.shape[1]:(h + 1) * a.shape[1], h].set(a[h])
        return m
    Abd1 = jnp.concatenate([blockdiag(a_src1), blockdiag(a_dst1)], axis=1)
    Abd2 = jnp.concatenate([blockdiag(a_src2), blockdiag(a_dst2)], axis=1)
    Ae1 = (We1.reshape(16, HEADS, 32) * a_edge1[None]).sum(-1)
    Ae2 = (We2.reshape(16, HEADS, 32) * a_edge2[None]).sum(-1)
    AeB = jnp.concatenate([Ae1, Ae2], axis=1)

    full = lambda shp: pl.BlockSpec(shp, lambda i: (0, 0))
    rows = lambda w: pl.BlockSpec((NBLK, w), lambda i: (i, 0))
    erows = lambda w: pl.BlockSpec((EBLK, w), lambda i: (i, 0))

    t1, ald1 = pl.pallas_call(
        _node_prep_body,
        grid=(NGRID,),
        in_specs=[rows(48), full((48, 128)), full((128, 2 * HEADS))],
        out_specs=(rows(TW), rows(HEADS)),
        out_shape=(jax.ShapeDtypeStruct((NP, TW), f32),
                   jax.ShapeDtypeStruct((NP, HEADS), f32)),
    )(x, W1p, Abd1)

    ale1, ale2 = pl.pallas_call(
        _edge_prep_body,
        grid=(EGRID,),
        in_specs=[erows(16), full((16, 2 * HEADS))],
        out_specs=(erows(HEADS), erows(HEADS)),
        out_shape=(jax.ShapeDtypeStruct((E, HEADS), f32),
                   jax.ShapeDtypeStruct((E, HEADS), f32)),
    )(edge_attr, AeB)

    zeros = jnp.zeros((ROWS_T, AW), f32)
    acc1 = _edge_pass(dst, src, ale1.reshape(-1), t1, _pad8(ald1), zeros)

    t2, ald2 = pl.pallas_call(
        _mid_body,
        grid=(NGRID,),
        in_specs=[rows(AW), pl.BlockSpec((1, 128), lambda i: (0, 0)),
                  full((128, 128)), full((128, 2 * HEADS))],
        out_specs=(rows(TW), rows(HEADS)),
        out_shape=(jax.ShapeDtypeStruct((NP, TW), f32),
                   jax.ShapeDtypeStruct((NP, HEADS), f32)),
    )(acc1, b1.reshape(1, 128), W2, Abd2)

    acc2 = _edge_pass(dst, src, ale2.reshape(-1), t2, _pad8(ald2), zeros)

    pred = pl.pallas_call(
        _head_body,
        grid=(NGRID,),
        in_specs=[rows(AW), pl.BlockSpec((1, 32), lambda i: (0, 0)),
                  full((32, 1)), pl.BlockSpec((1, 1), lambda i: (0, 0))],
        out_specs=rows(1),
        out_shape=jax.ShapeDtypeStruct((NP, 1), f32),
    )(acc2, b2.reshape(1, 32), Wh, bh.reshape(1, 1))

    return pred[:N]
